# baseline (speedup 1.0000x reference)
import jax
import jax.numpy as jnp
from jax.experimental import pallas as pl
from jax.experimental.pallas import tpu as pltpu

_TEMPERATURE = 0.35
_CHUNK = 2048


def _compute_chunk(xb, slots, usage, o_ref):
    s_sq = jnp.sum(slots * slots, axis=-1, keepdims=True)
    slots_n = (slots * jax.lax.rsqrt(jnp.maximum(s_sq, 1e-24))).astype(
        jnp.bfloat16)

    xb16 = xb.astype(jnp.bfloat16)
    x_sq = jnp.sum((xb16 * xb16).astype(jnp.float32), axis=-1, keepdims=True)
    inv_xn = jax.lax.rsqrt(jnp.maximum(x_sq, 1e-24))

    scores = jax.lax.dot_general(
        xb16, slots_n, (((1,), (1,)), ((), ())),
        preferred_element_type=jnp.float32)
    scores = scores * (inv_xn * (1.0 / _TEMPERATURE))

    e = jnp.exp(scores)
    w = (e / jnp.sum(e, axis=-1, keepdims=True)).astype(jnp.bfloat16)

    att = jax.lax.dot_general(
        w, slots.astype(jnp.bfloat16), (((1,), (0,)), ((), ())),
        preferred_element_type=jnp.float32)

    gate = (jnp.sum(usage) > 0).astype(jnp.float32)
    o_ref[...] = att * gate


def _bank_kernel(x_hbm, slots_hbm, usage_hbm, out_hbm,
                 inbuf, outbuf, slots_v, usage_v,
                 insem, outsem, ssem, usem):
    i = pl.program_id(0)
    nsteps = pl.num_programs(0)
    slot = jax.lax.rem(i, 2)

    def in_copy(j, s):
        return pltpu.make_async_copy(
            x_hbm.at[pl.ds(j * _CHUNK, _CHUNK), :], inbuf.at[s], insem.at[s])

    def out_copy(j, s):
        return pltpu.make_async_copy(
            outbuf.at[s], out_hbm.at[pl.ds(j * _CHUNK, _CHUNK), :],
            outsem.at[s])

    @pl.when(i == 0)
    def _():
        pltpu.make_async_copy(slots_hbm, slots_v, ssem).start()
        pltpu.make_async_copy(usage_hbm, usage_v, usem).start()
        in_copy(0, 0).start()

    @pl.when(i + 1 < nsteps)
    def _():
        in_copy(i + 1, 1 - slot).start()

    @pl.when(i >= 2)
    def _():
        out_copy(i - 2, slot).wait()

    @pl.when(i == 0)
    def _():
        pltpu.make_async_copy(slots_hbm, slots_v, ssem).wait()
        pltpu.make_async_copy(usage_hbm, usage_v, usem).wait()

    in_copy(i, slot).wait()

    _compute_chunk(inbuf[slot], slots_v[...], usage_v[...], outbuf.at[slot])

    out_copy(i, slot).start()

    @pl.when(i == nsteps - 1)
    def _():
        @pl.when(nsteps >= 2)
        def _():
            out_copy(i - 1, 1 - slot).wait()
        out_copy(i, slot).wait()


def kernel(x, slots, usage):
    b, l, d = x.shape
    s = slots.shape[0]
    n = b * l
    x2 = x.reshape(n, d)
    usage2 = usage.reshape(1, s)
    nsteps = n // _CHUNK

    out = pl.pallas_call(
        _bank_kernel,
        grid=(nsteps,),
        in_specs=[
            pl.BlockSpec(memory_space=pl.ANY),
            pl.BlockSpec(memory_space=pl.ANY),
            pl.BlockSpec(memory_space=pl.ANY),
        ],
        out_specs=pl.BlockSpec(memory_space=pl.ANY),
        out_shape=jax.ShapeDtypeStruct((n, d), jnp.float32),
        scratch_shapes=[
            pltpu.VMEM((2, _CHUNK, d), jnp.float32),
            pltpu.VMEM((2, _CHUNK, d), jnp.float32),
            pltpu.VMEM((s, d), jnp.float32),
            pltpu.VMEM((1, s), jnp.float32),
            pltpu.SemaphoreType.DMA((2,)),
            pltpu.SemaphoreType.DMA((2,)),
            pltpu.SemaphoreType.DMA,
            pltpu.SemaphoreType.DMA,
        ],
        compiler_params=pltpu.CompilerParams(
            dimension_semantics=("arbitrary",)),
    )(x2, slots, usage2)

    return out.reshape(b, l, d)


# speedup vs baseline: 1.0591x; 1.0591x over previous
import jax
import jax.numpy as jnp
from jax.experimental import pallas as pl
from jax.experimental.pallas import tpu as pltpu

_TEMPERATURE = 0.35
_CHUNK = 2048


def _compute_chunk(xb, slots, usage, o_ref):
    s_sq = jnp.sum(slots * slots, axis=-1, keepdims=True)
    slots_n = (slots * jax.lax.rsqrt(jnp.maximum(s_sq, 1e-24))).astype(
        jnp.bfloat16)

    xb16 = xb.astype(jnp.bfloat16)
    x_sq = jnp.sum((xb16 * xb16).astype(jnp.float32), axis=-1, keepdims=True)
    inv_xn = jax.lax.rsqrt(jnp.maximum(x_sq, 1e-24))

    scores = jax.lax.dot_general(
        xb16, slots_n, (((1,), (1,)), ((), ())),
        preferred_element_type=jnp.float32)
    scores = scores * (inv_xn * (1.0 / _TEMPERATURE))

    e = jnp.exp(scores)
    w = (e / jnp.sum(e, axis=-1, keepdims=True)).astype(jnp.bfloat16)

    att = jax.lax.dot_general(
        w, slots.astype(jnp.bfloat16), (((1,), (0,)), ((), ())),
        preferred_element_type=jnp.float32)

    gate = (jnp.sum(usage) > 0).astype(jnp.float32)
    o_ref[...] = att * gate


def _bank_kernel(x_hbm, slots_hbm, usage_hbm, out_hbm,
                 inbuf, outbuf, slots_v, usage_v,
                 insem, outsem, ssem, usem):
    i = pl.program_id(0)
    nsteps = pl.num_programs(0)
    islot = jax.lax.rem(i, 3)
    slot = jax.lax.rem(i, 2)

    def in_copy(j, s):
        return pltpu.make_async_copy(
            x_hbm.at[pl.ds(j * _CHUNK, _CHUNK), :], inbuf.at[s], insem.at[s])

    def out_copy(j, s):
        return pltpu.make_async_copy(
            outbuf.at[s], out_hbm.at[pl.ds(j * _CHUNK, _CHUNK), :],
            outsem.at[s])

    @pl.when(i == 0)
    def _():
        pltpu.make_async_copy(slots_hbm, slots_v, ssem).start()
        pltpu.make_async_copy(usage_hbm, usage_v, usem).start()
        in_copy(0, 0).start()
        in_copy(1, 1).start()

    @pl.when(i + 2 < nsteps)
    def _():
        in_copy(i + 2, jax.lax.rem(i + 2, 3)).start()

    @pl.when(i >= 2)
    def _():
        out_copy(i - 2, slot).wait()

    @pl.when(i == 0)
    def _():
        pltpu.make_async_copy(slots_hbm, slots_v, ssem).wait()
        pltpu.make_async_copy(usage_hbm, usage_v, usem).wait()

    in_copy(i, islot).wait()

    _compute_chunk(inbuf[islot], slots_v[...], usage_v[...], outbuf.at[slot])

    out_copy(i, slot).start()

    @pl.when(i == nsteps - 1)
    def _():
        @pl.when(nsteps >= 2)
        def _():
            out_copy(i - 1, 1 - slot).wait()
        out_copy(i, slot).wait()


def kernel(x, slots, usage):
    b, l, d = x.shape
    s = slots.shape[0]
    n = b * l
    x2 = x.reshape(n, d)
    usage2 = usage.reshape(1, s)
    nsteps = n // _CHUNK

    out = pl.pallas_call(
        _bank_kernel,
        grid=(nsteps,),
        in_specs=[
            pl.BlockSpec(memory_space=pl.ANY),
            pl.BlockSpec(memory_space=pl.ANY),
            pl.BlockSpec(memory_space=pl.ANY),
        ],
        out_specs=pl.BlockSpec(memory_space=pl.ANY),
        out_shape=jax.ShapeDtypeStruct((n, d), jnp.float32),
        scratch_shapes=[
            pltpu.VMEM((3, _CHUNK, d), jnp.float32),
            pltpu.VMEM((2, _CHUNK, d), jnp.float32),
            pltpu.VMEM((s, d), jnp.float32),
            pltpu.VMEM((1, s), jnp.float32),
            pltpu.SemaphoreType.DMA((3,)),
            pltpu.SemaphoreType.DMA((2,)),
            pltpu.SemaphoreType.DMA,
            pltpu.SemaphoreType.DMA,
        ],
        compiler_params=pltpu.CompilerParams(
            dimension_semantics=("arbitrary",)),
    )(x2, slots, usage2)

    return out.reshape(b, l, d)


# speedup vs baseline: 1.0934x; 1.0324x over previous
import jax
import jax.numpy as jnp
from jax.experimental import pallas as pl
from jax.experimental.pallas import tpu as pltpu

_TEMPERATURE = 0.35
_CHUNK = 2048


def _compute_chunk(xb, slots, usage, o_ref):
    s_sq = jnp.sum(slots * slots, axis=-1, keepdims=True)
    slots_n = (slots * jax.lax.rsqrt(jnp.maximum(s_sq, 1e-24))).astype(
        jnp.bfloat16)

    xb16 = xb.astype(jnp.bfloat16)
    x_sq = jnp.sum((xb16 * xb16).astype(jnp.float32), axis=-1, keepdims=True)
    inv_xn = jax.lax.rsqrt(jnp.maximum(x_sq, 1e-24))

    scores = jax.lax.dot_general(
        xb16, slots_n, (((1,), (1,)), ((), ())),
        preferred_element_type=jnp.float32)
    scores = scores * (inv_xn * (1.0 / _TEMPERATURE))

    e = jnp.exp(scores)
    w = (e / jnp.sum(e, axis=-1, keepdims=True)).astype(jnp.bfloat16)

    att = jax.lax.dot_general(
        w, slots.astype(jnp.bfloat16), (((1,), (0,)), ((), ())),
        preferred_element_type=jnp.float32)

    gate = (jnp.sum(usage) > 0).astype(jnp.float32)
    o_ref[...] = att * gate


def _bank_kernel(x_hbm, slots_hbm, usage_hbm, out_hbm,
                 inbuf, outbuf, slots_v, usage_v,
                 insem, outsem, ssem, usem):
    i = pl.program_id(0)
    nsteps = pl.num_programs(0)
    slot = jax.lax.rem(i, 3)

    def in_copy(j, s):
        return pltpu.make_async_copy(
            x_hbm.at[pl.ds(j * _CHUNK, _CHUNK), :], inbuf.at[s], insem.at[s])

    def out_copy(j, s):
        return pltpu.make_async_copy(
            outbuf.at[s], out_hbm.at[pl.ds(j * _CHUNK, _CHUNK), :],
            outsem.at[s])

    @pl.when(i == 0)
    def _():
        pltpu.make_async_copy(slots_hbm, slots_v, ssem).start()
        pltpu.make_async_copy(usage_hbm, usage_v, usem).start()
        in_copy(0, 0).start()
        in_copy(1, 1).start()

    @pl.when(i + 2 < nsteps)
    def _():
        in_copy(i + 2, jax.lax.rem(i + 2, 3)).start()

    @pl.when(i >= 3)
    def _():
        out_copy(i - 3, slot).wait()

    @pl.when(i == 0)
    def _():
        pltpu.make_async_copy(slots_hbm, slots_v, ssem).wait()
        pltpu.make_async_copy(usage_hbm, usage_v, usem).wait()

    in_copy(i, slot).wait()

    _compute_chunk(inbuf[slot], slots_v[...], usage_v[...], outbuf.at[slot])

    out_copy(i, slot).start()

    @pl.when(i == nsteps - 1)
    def _():
        @pl.when(nsteps >= 3)
        def _():
            out_copy(i - 2, jax.lax.rem(i - 2, 3)).wait()
        @pl.when(nsteps >= 2)
        def _():
            out_copy(i - 1, jax.lax.rem(i - 1, 3)).wait()
        out_copy(i, slot).wait()


def kernel(x, slots, usage):
    b, l, d = x.shape
    s = slots.shape[0]
    n = b * l
    x2 = x.reshape(n, d)
    usage2 = usage.reshape(1, s)
    nsteps = n // _CHUNK

    out = pl.pallas_call(
        _bank_kernel,
        grid=(nsteps,),
        in_specs=[
            pl.BlockSpec(memory_space=pl.ANY),
            pl.BlockSpec(memory_space=pl.ANY),
            pl.BlockSpec(memory_space=pl.ANY),
        ],
        out_specs=pl.BlockSpec(memory_space=pl.ANY),
        out_shape=jax.ShapeDtypeStruct((n, d), jnp.float32),
        scratch_shapes=[
            pltpu.VMEM((3, _CHUNK, d), jnp.float32),
            pltpu.VMEM((3, _CHUNK, d), jnp.float32),
            pltpu.VMEM((s, d), jnp.float32),
            pltpu.VMEM((1, s), jnp.float32),
            pltpu.SemaphoreType.DMA((3,)),
            pltpu.SemaphoreType.DMA((3,)),
            pltpu.SemaphoreType.DMA,
            pltpu.SemaphoreType.DMA,
        ],
        compiler_params=pltpu.CompilerParams(
            dimension_semantics=("arbitrary",)),
    )(x2, slots, usage2)

    return out.reshape(b, l, d)


# speedup vs baseline: 1.0943x; 1.0008x over previous
import jax
import jax.numpy as jnp
from jax.experimental import pallas as pl
from jax.experimental.pallas import tpu as pltpu

_TEMPERATURE = 0.35
_CHUNK = 2048


def _compute_chunk(xb, slots, usage, o_ref):
    s_sq = jnp.sum(slots * slots, axis=-1, keepdims=True)
    slots_n = (slots * jax.lax.rsqrt(jnp.maximum(s_sq, 1e-24))).astype(
        jnp.bfloat16)

    xb16 = xb.astype(jnp.bfloat16)
    x_sq = jnp.sum((xb16 * xb16).astype(jnp.float32), axis=-1, keepdims=True)
    inv_xn = jax.lax.rsqrt(jnp.maximum(x_sq, 1e-24))

    scores = jax.lax.dot_general(
        xb16, slots_n, (((1,), (1,)), ((), ())),
        preferred_element_type=jnp.float32)
    scores = scores * (inv_xn * (1.0 / _TEMPERATURE))

    e = jnp.exp(scores)
    w = (e / jnp.sum(e, axis=-1, keepdims=True)).astype(jnp.bfloat16)

    att = jax.lax.dot_general(
        w, slots.astype(jnp.bfloat16), (((1,), (0,)), ((), ())),
        preferred_element_type=jnp.float32)

    gate = (jnp.sum(usage) > 0).astype(jnp.float32)
    o_ref[...] = att * gate


def _bank_kernel(x_hbm, slots_hbm, usage_hbm, out_hbm,
                 inbuf, outbuf, slots_v, usage_v,
                 insem, outsem, ssem, usem):
    i = pl.program_id(0)
    nsteps = pl.num_programs(0)
    slot = jax.lax.rem(i, 3)
    oslot = jax.lax.rem(i, 4)

    def in_copy(j, s):
        return pltpu.make_async_copy(
            x_hbm.at[pl.ds(j * _CHUNK, _CHUNK), :], inbuf.at[s], insem.at[s])

    def out_copy(j, s):
        return pltpu.make_async_copy(
            outbuf.at[s], out_hbm.at[pl.ds(j * _CHUNK, _CHUNK), :],
            outsem.at[s])

    @pl.when(i == 0)
    def _():
        pltpu.make_async_copy(slots_hbm, slots_v, ssem).start()
        pltpu.make_async_copy(usage_hbm, usage_v, usem).start()
        in_copy(0, 0).start()
        in_copy(1, 1).start()

    @pl.when(i + 2 < nsteps)
    def _():
        in_copy(i + 2, jax.lax.rem(i + 2, 3)).start()

    @pl.when(i >= 4)
    def _():
        out_copy(i - 4, oslot).wait()

    @pl.when(i == 0)
    def _():
        pltpu.make_async_copy(slots_hbm, slots_v, ssem).wait()
        pltpu.make_async_copy(usage_hbm, usage_v, usem).wait()

    in_copy(i, slot).wait()

    _compute_chunk(inbuf[slot], slots_v[...], usage_v[...], outbuf.at[oslot])

    out_copy(i, oslot).start()

    @pl.when(i == nsteps - 1)
    def _():
        @pl.when(nsteps >= 4)
        def _():
            out_copy(i - 3, jax.lax.rem(i - 3, 4)).wait()
        @pl.when(nsteps >= 3)
        def _():
            out_copy(i - 2, jax.lax.rem(i - 2, 4)).wait()
        @pl.when(nsteps >= 2)
        def _():
            out_copy(i - 1, jax.lax.rem(i - 1, 4)).wait()
        out_copy(i, oslot).wait()


def kernel(x, slots, usage):
    b, l, d = x.shape
    s = slots.shape[0]
    n = b * l
    x2 = x.reshape(n, d)
    usage2 = usage.reshape(1, s)
    nsteps = n // _CHUNK

    out = pl.pallas_call(
        _bank_kernel,
        grid=(nsteps,),
        in_specs=[
            pl.BlockSpec(memory_space=pl.ANY),
            pl.BlockSpec(memory_space=pl.ANY),
            pl.BlockSpec(memory_space=pl.ANY),
        ],
        out_specs=pl.BlockSpec(memory_space=pl.ANY),
        out_shape=jax.ShapeDtypeStruct((n, d), jnp.float32),
        scratch_shapes=[
            pltpu.VMEM((3, _CHUNK, d), jnp.float32),
            pltpu.VMEM((4, _CHUNK, d), jnp.float32),
            pltpu.VMEM((s, d), jnp.float32),
            pltpu.VMEM((1, s), jnp.float32),
            pltpu.SemaphoreType.DMA((3,)),
            pltpu.SemaphoreType.DMA((4,)),
            pltpu.SemaphoreType.DMA,
            pltpu.SemaphoreType.DMA,
        ],
        compiler_params=pltpu.CompilerParams(
            dimension_semantics=("arbitrary",),
            vmem_limit_bytes=63 * 1024 * 1024),
    )(x2, slots, usage2)

    return out.reshape(b, l, d)


# speedup vs baseline: 1.0982x; 1.0035x over previous
import jax
import jax.numpy as jnp
from jax.experimental import pallas as pl
from jax.experimental.pallas import tpu as pltpu

_TEMPERATURE = 0.35
_CHUNK = 2048


def _compute_chunk(xb, slots, usage, o_ref):
    s_sq = jnp.sum(slots * slots, axis=-1, keepdims=True)
    slots_n = (slots * jax.lax.rsqrt(jnp.maximum(s_sq, 1e-24))).astype(
        jnp.bfloat16)

    xb16 = xb.astype(jnp.bfloat16)
    x_sq = jnp.sum((xb16 * xb16).astype(jnp.float32), axis=-1, keepdims=True)
    inv_xn = jax.lax.rsqrt(jnp.maximum(x_sq, 1e-24))

    scores = jax.lax.dot_general(
        xb16, slots_n, (((1,), (1,)), ((), ())),
        preferred_element_type=jnp.float32)
    scores = scores * (inv_xn * (1.0 / _TEMPERATURE))

    e = jnp.exp(scores)
    w = (e / jnp.sum(e, axis=-1, keepdims=True)).astype(jnp.bfloat16)

    att = jax.lax.dot_general(
        w, slots.astype(jnp.bfloat16), (((1,), (0,)), ((), ())),
        preferred_element_type=jnp.float32)

    gate = (jnp.sum(usage) > 0).astype(jnp.float32)
    o_ref[...] = att * gate


def _bank_kernel(x_hbm, slots_hbm, usage_hbm, out_hbm,
                 inbuf, outbuf, slots_v, usage_v,
                 insem, outsem, ssem, usem):
    i = pl.program_id(0)
    nsteps = pl.num_programs(0)
    slot = jax.lax.rem(i, 3)

    def in_copy(j, s):
        return pltpu.make_async_copy(
            x_hbm.at[pl.ds(j * _CHUNK, _CHUNK), :], inbuf.at[s], insem.at[s])

    def out_copy(j, s):
        return pltpu.make_async_copy(
            outbuf.at[s], out_hbm.at[pl.ds(j * _CHUNK, _CHUNK), :],
            outsem.at[s])

    @pl.when(i == 0)
    def _():
        pltpu.make_async_copy(slots_hbm, slots_v, ssem).start()
        pltpu.make_async_copy(usage_hbm, usage_v, usem).start()
        in_copy(0, 0).start()
        in_copy(1, 1).start()

    @pl.when(i + 2 < nsteps)
    def _():
        in_copy(i + 2, jax.lax.rem(i + 2, 3)).start()

    @pl.when(i >= 3)
    def _():
        out_copy(i - 3, slot).wait()

    @pl.when(i == 0)
    def _():
        pltpu.make_async_copy(slots_hbm, slots_v, ssem).wait()
        pltpu.make_async_copy(usage_hbm, usage_v, usem).wait()

    in_copy(i, slot).wait()

    _compute_chunk(inbuf[slot], slots_v[...], usage_v[...], outbuf.at[slot])

    out_copy(i, slot).start()

    @pl.when(i == nsteps - 1)
    def _():
        @pl.when(nsteps >= 3)
        def _():
            out_copy(i - 2, jax.lax.rem(i - 2, 3)).wait()
        @pl.when(nsteps >= 2)
        def _():
            out_copy(i - 1, jax.lax.rem(i - 1, 3)).wait()
        out_copy(i, slot).wait()


def kernel(x, slots, usage):
    b, l, d = x.shape
    s = slots.shape[0]
    n = b * l
    x2 = x.reshape(n, d)
    usage2 = usage.reshape(1, s)
    nsteps = n // _CHUNK

    out = pl.pallas_call(
        _bank_kernel,
        grid=(nsteps,),
        in_specs=[
            pl.BlockSpec(memory_space=pl.ANY),
            pl.BlockSpec(memory_space=pl.ANY),
            pl.BlockSpec(memory_space=pl.ANY),
        ],
        out_specs=pl.BlockSpec(memory_space=pl.ANY),
        out_shape=jax.ShapeDtypeStruct((n, d), jnp.float32),
        scratch_shapes=[
            pltpu.VMEM((3, _CHUNK, d), jnp.float32),
            pltpu.VMEM((3, _CHUNK, d), jnp.float32),
            pltpu.VMEM((s, d), jnp.float32),
            pltpu.VMEM((1, s), jnp.float32),
            pltpu.SemaphoreType.DMA((3,)),
            pltpu.SemaphoreType.DMA((3,)),
            pltpu.SemaphoreType.DMA,
            pltpu.SemaphoreType.DMA,
        ],
        compiler_params=pltpu.CompilerParams(
            dimension_semantics=("arbitrary",)),
    )(x2, slots, usage2)

    return out.reshape(b, l, d)
